# baseline (speedup 1.0000x reference)
import jax
import jax.numpy as jnp
import numpy as np
from jax.experimental import pallas as pl
from jax.experimental.pallas import tpu as pltpu

B, A, NC, REG_MAX = 16, 33600, 80, 16
N = B * A
D = 4 * (REG_MAX + 1)
BINS = REG_MAX + 1
NCQ = NC // 4
EPS = 1e-07


def _atan(x):
    big = x > 2.414213562373095
    mid = x > 0.4142135623730950
    t = jnp.where(big, -1.0 / x, jnp.where(mid, (x - 1.0) / (x + 1.0), x))
    base = jnp.where(big, np.float32(np.pi / 2),
                     jnp.where(mid, np.float32(np.pi / 4), 0.0))
    z = t * t
    p = (((8.05374449538e-2 * z - 1.38776856032e-1) * z
          + 1.99777106478e-1) * z - 3.33329491539e-1) * z * t + t
    return base + p


def _loss_body(pd_ref, ts_ref, pb_ref, tb_ref, ap_ref, mk_ref, out_ref,
               ts_acc, dfl_acc, cc_acc):
    b = pl.program_id(0)
    s = pl.program_id(1)

    @pl.when((b == 0) & (s == 0))
    def _init():
        out_ref[...] = jnp.zeros_like(out_ref)

    @pl.when(s == 0)
    def _per_batch_init():
        pb = pb_ref[...].reshape(4, A)
        tbx = tb_ref[...].reshape(4, A)
        b1_x1, b1_y1 = pb[0:1, :], pb[1:2, :]
        b1_x2, b1_y2 = pb[2:3, :], pb[3:4, :]
        b2_x1, b2_y1 = tbx[0:1, :], tbx[1:2, :]
        b2_x2, b2_y2 = tbx[2:3, :], tbx[3:4, :]
        w1, h1 = b1_x2 - b1_x1, b1_y2 - b1_y1 + EPS
        w2, h2 = b2_x2 - b2_x1, b2_y2 - b2_y1 + EPS
        inter = (jnp.clip(jnp.minimum(b1_x2, b2_x2) - jnp.maximum(b1_x1, b2_x1), 0.0, None)
                 * jnp.clip(jnp.minimum(b1_y2, b2_y2) - jnp.maximum(b1_y1, b2_y1), 0.0, None))
        union = w1 * h1 + w2 * h2 - inter + EPS
        cent = ((b2_x1 + b2_x2 - b1_x1 - b1_x2) ** 2
                + (b2_y1 + b2_y2 - b1_y1 - b1_y2) ** 2) / 4.0
        iou = inter / union
        cw = jnp.maximum(b1_x2, b2_x2) - jnp.minimum(b1_x1, b2_x1)
        ch = jnp.maximum(b1_y2, b2_y2) - jnp.minimum(b1_y1, b2_y1)
        c2 = cw ** 2 + ch ** 2 + EPS
        v = (4.0 / np.pi ** 2) * (_atan(w2 / h2) - _atan(w1 / h1)) ** 2
        alpha = v / (v - iou + (1.0 + EPS))
        ciou = iou - (cent / c2 + v * alpha)
        cc_acc[0:1, :] = 1.0 - ciou
        cc_acc[1:2, :] = cent

    ts = ts_ref[...].reshape(NCQ, A)
    ones_q = jnp.ones((1, NCQ), jnp.float32)
    tpart = jax.lax.dot_general(ones_q, ts, (((1,), (0,)), ((), ())),
                                preferred_element_type=jnp.float32)
    prev_t = jnp.where(s == 0, jnp.zeros((1, A), jnp.float32), ts_acc[...])
    ts_acc[...] = prev_t + tpart

    pd = pd_ref[...].reshape(BINS, A)
    tb = tb_ref[...].reshape(4, A)
    ap = ap_ref[...]
    ax, ay = ap[0:1, :], ap[1:2, :]
    ltrb_row = jnp.where(
        s == 0, ax - tb[0:1, :],
        jnp.where(s == 1, ay - tb[1:2, :],
                  jnp.where(s == 2, tb[2:3, :] - ax, tb[3:4, :] - ay)))
    ltrb_row = jnp.clip(ltrb_row, 0.0, REG_MAX - 0.01)
    jcol = jax.lax.broadcasted_iota(jnp.int32, (BINS, 1), 0).astype(jnp.float32)
    u = jnp.maximum(1.0 - jnp.abs(jcol - ltrb_row), 0.0)
    ones_b = jnp.ones((1, BINS), jnp.float32)
    gsum = jax.lax.dot_general(ones_b, pd * u, (((1,), (0,)), ((), ())),
                               preferred_element_type=jnp.float32)
    esum = jax.lax.dot_general(ones_b, jnp.exp(pd), (((1,), (0,)), ((), ())),
                               preferred_element_type=jnp.float32)
    term = 0.25 * (jnp.log(esum) - gsum)
    prev_d = jnp.where(s == 0, jnp.zeros((1, A), jnp.float32), dfl_acc[...])
    dfl_acc[...] = prev_d + term

    @pl.when(s == 3)
    def _finish():
        mask = mk_ref[...].reshape(1, A)
        weight = ts_acc[...] * mask
        ones_a = jnp.ones((A, 1), jnp.float32)

        def rsum(row):
            return jax.lax.dot_general(row, ones_a, (((1,), (0,)), ((), ())),
                                       preferred_element_type=jnp.float32)[0, 0]

        s_iou = rsum(cc_acc[0:1, :] * weight)
        s_cent = rsum(cc_acc[1:2, :] * weight)
        s_dfl = rsum(dfl_acc[...] * weight)
        lane128 = jax.lax.broadcasted_iota(jnp.int32, (1, 128), 1)
        row = (jnp.where(lane128 == 0, s_iou, 0.0)
               + jnp.where(lane128 == 1, s_dfl, 0.0)
               + jnp.where(lane128 == 2, s_cent, 0.0))
        out_ref[...] += row


def kernel(pred_dist, pred_bboxes, anchor_points, target_bboxes,
           target_scores, target_scores_sum, fg_mask):
    pd4 = jnp.transpose(pred_dist, (2, 0, 1)).reshape(D, B, 1, A)
    ts4 = jnp.transpose(target_scores, (0, 2, 1)).reshape(B, NC, 1, A)
    pb4 = jnp.transpose(pred_bboxes, (0, 2, 1)).reshape(B, 4, 1, A)
    tb4 = jnp.transpose(target_bboxes, (0, 2, 1)).reshape(B, 4, 1, A)
    apt = anchor_points.T
    mk = fg_mask.astype(jnp.float32).reshape(B, 1, A)

    sums = pl.pallas_call(
        _loss_body,
        grid=(B, 4),
        in_specs=[
            pl.BlockSpec((BINS, 1, 1, A), lambda b, s: (s, b, 0, 0)),
            pl.BlockSpec((1, NCQ, 1, A), lambda b, s: (b, s, 0, 0)),
            pl.BlockSpec((1, 4, 1, A), lambda b, s: (b, 0, 0, 0)),
            pl.BlockSpec((1, 4, 1, A), lambda b, s: (b, 0, 0, 0)),
            pl.BlockSpec((2, A), lambda b, s: (0, 0)),
            pl.BlockSpec((1, 1, A), lambda b, s: (b, 0, 0)),
        ],
        out_specs=pl.BlockSpec((1, 128), lambda b, s: (0, 0)),
        out_shape=jax.ShapeDtypeStruct((1, 128), jnp.float32),
        scratch_shapes=[
            pltpu.VMEM((1, A), jnp.float32),
            pltpu.VMEM((1, A), jnp.float32),
            pltpu.VMEM((2, A), jnp.float32),
        ],
    )(pd4, ts4, pb4, tb4, apt, mk)

    inv = 1.0 / target_scores_sum
    return (sums[0, 0] * inv, sums[0, 1] * inv, sums[0, 2] * inv)


# speedup vs baseline: 1.0783x; 1.0783x over previous
import jax
import jax.numpy as jnp
import numpy as np
from jax.experimental import pallas as pl

B, A, NC, REG_MAX = 16, 33600, 80, 16
N = B * A
D = 4 * (REG_MAX + 1)
AB = 21504
EPS = 1e-07


def _atan(x):
    big = x > 2.414213562373095
    mid = x > 0.4142135623730950
    t = jnp.where(big, -1.0 / x, jnp.where(mid, (x - 1.0) / (x + 1.0), x))
    base = jnp.where(big, np.float32(np.pi / 2),
                     jnp.where(mid, np.float32(np.pi / 4), 0.0))
    z = t * t
    p = (((8.05374449538e-2 * z - 1.38776856032e-1) * z
          + 1.99777106478e-1) * z - 3.33329491539e-1) * z * t + t
    return base + p


def _consts():
    bins = REG_MAX + 1
    rr = jax.lax.broadcasted_iota(jnp.int32, (D, 6), 1)
    ss = jax.lax.broadcasted_iota(jnp.int32, (D, 6), 0) // bins
    plus = (((rr == 4) & (ss == 0)) | ((rr == 5) & (ss == 1))
            | ((rr == 2) & (ss == 2)) | ((rr == 3) & (ss == 3)))
    minus = (((rr == 0) & (ss == 0)) | ((rr == 1) & (ss == 1))
             | ((rr == 4) & (ss == 2)) | ((rr == 5) & (ss == 3)))
    m6t = plus.astype(jnp.float32) - minus.astype(jnp.float32)
    drow = jax.lax.broadcasted_iota(jnp.int32, (D, 1), 0)
    jcol = (drow - (drow // bins) * bins).astype(jnp.float32)
    lr = jax.lax.broadcasted_iota(jnp.int32, (4, D), 1) // bins
    lc = jax.lax.broadcasted_iota(jnp.int32, (4, D), 0)
    r4t = (lr == lc).astype(jnp.float32)
    return m6t, jcol, r4t


def _tsum_body(ts_ref, out_ref):
    t = ts_ref[...].reshape(NC, A)
    ones = jnp.ones((1, NC), jnp.float32)
    s = jax.lax.dot_general(ones, t, (((1,), (0,)), ((), ())),
                            preferred_element_type=jnp.float32)
    out_ref[...] = s.reshape(1, 1, A)


def _loss_body(pd_ref, thin_ref, w_ref, out_ref):
    i = pl.program_id(0)

    @pl.when(i == 0)
    def _init():
        out_ref[...] = jnp.zeros_like(out_ref)

    thin = thin_ref[...]
    b1_x1, b1_y1 = thin[0:1, :], thin[1:2, :]
    b1_x2, b1_y2 = thin[2:3, :], thin[3:4, :]
    b2_x1, b2_y1 = thin[4:5, :], thin[5:6, :]
    b2_x2, b2_y2 = thin[6:7, :], thin[7:8, :]
    mask = thin[10:11, :]
    weight = w_ref[...] * mask
    w1, h1 = b1_x2 - b1_x1, b1_y2 - b1_y1 + EPS
    w2, h2 = b2_x2 - b2_x1, b2_y2 - b2_y1 + EPS
    inter = (jnp.clip(jnp.minimum(b1_x2, b2_x2) - jnp.maximum(b1_x1, b2_x1), 0.0, None)
             * jnp.clip(jnp.minimum(b1_y2, b2_y2) - jnp.maximum(b1_y1, b2_y1), 0.0, None))
    union = w1 * h1 + w2 * h2 - inter + EPS
    cent = ((b2_x1 + b2_x2 - b1_x1 - b1_x2) ** 2
            + (b2_y1 + b2_y2 - b1_y1 - b1_y2) ** 2) / 4.0
    iou = inter / union
    cw = jnp.maximum(b1_x2, b2_x2) - jnp.minimum(b1_x1, b2_x1)
    ch = jnp.maximum(b1_y2, b2_y2) - jnp.minimum(b1_y1, b2_y1)
    c2 = cw ** 2 + ch ** 2 + EPS
    v = (4.0 / np.pi ** 2) * (_atan(w2 / h2) - _atan(w1 / h1)) ** 2
    alpha = v / (v - iou + (1.0 + EPS))
    ciou = iou - (cent / c2 + v * alpha)

    pd = pd_ref[...]
    thin6 = thin[4:10, :]
    m6t, jcol, r4t = _consts()
    ltrb = jax.lax.dot_general(m6t, thin6, (((1,), (0,)), ((), ())),
                               preferred_element_type=jnp.float32)
    ltrb = jnp.clip(ltrb, 0.0, REG_MAX - 0.01)
    u = jnp.maximum(1.0 - jnp.abs(jcol - ltrb), 0.0)
    ones_d = jnp.ones((1, D), jnp.float32)
    gsum = jax.lax.dot_general(ones_d, pd * u, (((1,), (0,)), ((), ())),
                               preferred_element_type=jnp.float32)
    es4 = jax.lax.dot_general(r4t, jnp.exp(pd), (((1,), (0,)), ((), ())),
                              preferred_element_type=jnp.float32)
    lsum = jax.lax.dot_general(jnp.ones((1, 4), jnp.float32), jnp.log(es4),
                               (((1,), (0,)), ((), ())),
                               preferred_element_type=jnp.float32)
    dfl = 0.25 * (lsum - gsum)

    ones_ab = jnp.ones((AB, 1), jnp.float32)

    def rsum(row):
        return jax.lax.dot_general(row, ones_ab, (((1,), (0,)), ((), ())),
                                   preferred_element_type=jnp.float32)[0, 0]

    s_iou = rsum((1.0 - ciou) * weight)
    s_cent = rsum(cent * weight)
    s_dfl = rsum(dfl * weight)
    lane128 = jax.lax.broadcasted_iota(jnp.int32, (1, 128), 1)
    row = (jnp.where(lane128 == 0, s_iou, 0.0)
           + jnp.where(lane128 == 1, s_dfl, 0.0)
           + jnp.where(lane128 == 2, s_cent, 0.0))
    out_ref[...] += row


def kernel(pred_dist, pred_bboxes, anchor_points, target_bboxes,
           target_scores, target_scores_sum, fg_mask):
    pdt = jnp.transpose(pred_dist, (2, 0, 1)).reshape(D, N)
    ts3 = jnp.transpose(target_scores, (0, 2, 1))
    pbt = jnp.transpose(pred_bboxes, (2, 0, 1)).reshape(4, N)
    tbt = jnp.transpose(target_bboxes, (2, 0, 1)).reshape(4, N)
    apt = jnp.broadcast_to(anchor_points.T[:, None, :], (2, B, A)).reshape(2, N)
    mk = fg_mask.reshape(1, N).astype(jnp.float32)
    thin = jnp.concatenate([pbt, tbt, apt, mk], axis=0)

    tsum = pl.pallas_call(
        _tsum_body,
        grid=(B,),
        in_specs=[pl.BlockSpec((1, NC, A), lambda b: (b, 0, 0))],
        out_specs=pl.BlockSpec((1, 1, A), lambda b: (b, 0, 0)),
        out_shape=jax.ShapeDtypeStruct((B, 1, A), jnp.float32),
    )(ts3)
    wrow = tsum.reshape(1, N)

    sums = pl.pallas_call(
        _loss_body,
        grid=(N // AB,),
        in_specs=[
            pl.BlockSpec((D, AB), lambda i: (0, i)),
            pl.BlockSpec((11, AB), lambda i: (0, i)),
            pl.BlockSpec((1, AB), lambda i: (0, i)),
        ],
        out_specs=pl.BlockSpec((1, 128), lambda i: (0, 0)),
        out_shape=jax.ShapeDtypeStruct((1, 128), jnp.float32),
    )(pdt, thin, wrow)

    inv = 1.0 / target_scores_sum
    return (sums[0, 0] * inv, sums[0, 1] * inv, sums[0, 2] * inv)
